# baseline (speedup 1.0000x reference)
import jax
import jax.numpy as jnp
from jax import lax
from jax.experimental import pallas as pl
from jax.experimental.pallas import tpu as pltpu
from jax.experimental.pallas import tpu_sc as plsc

_DIM1 = 120000
_N = 256
_B = 2400
_ROWS = (3, 10, 12089)
_BCHUNK = _ROWS[2] // _B

_HEAD = 67200
_TAIL = _DIM1 - _HEAD
_NW = 24
_WROWS = _TAIL // _NW
_CHUNK = 440
_NCHUNK_SC = _WROWS // _CHUNK
_NC = 2


def _sc_tail_fill(out_hbm, zbuf, sem):
    wid = lax.axis_index("s") * _NC + lax.axis_index("c")
    zeros16 = jnp.zeros((16,), jnp.float32)

    def _zero_row(r, carry):
        for j in range(_N // 16):
            zbuf[r, pl.ds(j * 16, 16)] = zeros16
        return carry

    lax.fori_loop(0, _CHUNK, _zero_row, 0)

    @pl.when(wid < _NW)
    def _():
        base = _HEAD + wid * _WROWS
        copies = [
            pltpu.async_copy(zbuf, out_hbm.at[pl.ds(base + k * _CHUNK, _CHUNK)], sem)
            for k in range(_NCHUNK_SC)
        ]
        for cp in copies:
            cp.wait()


def _stage_tiles(row0_ref, zbuf, abuf, bbuf):
    row0 = row0_ref[...]
    ids = jax.lax.broadcasted_iota(jnp.int32, (_B, 1), 0)
    zbuf[...] = jnp.zeros((_B, _N), jnp.float32)
    abuf[...] = jnp.where((ids == _ROWS[0]) | (ids == _ROWS[1]), row0, 0.0)
    bbuf[...] = jnp.where(ids == _ROWS[2] - _BCHUNK * _B, row0, 0.0)


def _fire_chunks(out_ref, nchunk, zbuf, abuf, bbuf, sem):
    copies = []
    for k in range(nchunk):
        src = abuf if k == 0 else (bbuf if k == _BCHUNK else zbuf)
        copies.append(pltpu.make_async_copy(src, out_ref.at[pl.ds(k * _B, _B)], sem))
    for c in copies:
        c.start()
    for c in copies:
        c.wait()


def _tc_out1_body(row0_ref, out_ref, zbuf, abuf, bbuf, sem):
    _stage_tiles(row0_ref, zbuf, abuf, bbuf)
    _fire_chunks(out_ref, _DIM1 // _B, zbuf, abuf, bbuf, sem)


def _tc_head_body(row0_ref, _, out_ref, zbuf, abuf, bbuf, sem):
    _stage_tiles(row0_ref, zbuf, abuf, bbuf)
    _fire_chunks(out_ref, _HEAD // _B, zbuf, abuf, bbuf, sem)


def kernel(arr2):
    row0 = arr2[0:1, :]
    out_shape = jax.ShapeDtypeStruct((_DIM1, _N), jnp.float32)
    _scratch = [
        pltpu.VMEM((_B, _N), jnp.float32),
        pltpu.VMEM((_B, _N), jnp.float32),
        pltpu.VMEM((_B, _N), jnp.float32),
        pltpu.SemaphoreType.DMA,
    ]

    sc_tail = pl.kernel(
        _sc_tail_fill,
        mesh=plsc.VectorSubcoreMesh(core_axis_name="c", subcore_axis_name="s"),
        out_type=out_shape,
        scratch_types=[
            pltpu.VMEM((_CHUNK, _N), jnp.float32),
            pltpu.SemaphoreType.DMA,
        ],
    )
    out2_tail = sc_tail()

    out1 = pl.pallas_call(
        _tc_out1_body,
        in_specs=[pl.BlockSpec(memory_space=pltpu.VMEM)],
        out_specs=pl.BlockSpec(memory_space=pltpu.MemorySpace.HBM),
        out_shape=out_shape,
        scratch_shapes=_scratch,
    )(row0)

    out2 = pl.pallas_call(
        _tc_head_body,
        in_specs=[
            pl.BlockSpec(memory_space=pltpu.VMEM),
            pl.BlockSpec(memory_space=pltpu.MemorySpace.HBM),
        ],
        out_specs=pl.BlockSpec(memory_space=pltpu.MemorySpace.HBM),
        out_shape=out_shape,
        scratch_shapes=_scratch,
        input_output_aliases={1: 0},
    )(row0, out2_tail)

    return (out1, out2)


# speedup vs baseline: 1.2460x; 1.2460x over previous
import jax
import jax.numpy as jnp
from jax.experimental import pallas as pl

_DIM1 = 120000
_N = 256
_BLOCK = 2400
_GRID = _DIM1 // _BLOCK
_ROWS = (3, 10, 12089)


def _spmm_body(row0_ref, out1_ref, out2_ref):
    i = pl.program_id(0)
    ids = jax.lax.broadcasted_iota(jnp.int32, (_BLOCK, 1), 0) + i * _BLOCK
    mask = (ids == _ROWS[0]) | (ids == _ROWS[1]) | (ids == _ROWS[2])
    block = jnp.where(mask, row0_ref[...], 0.0)
    out1_ref[...] = block
    out2_ref[...] = block


def kernel(arr2):
    row0 = arr2[0:1, :]
    out_spec = pl.BlockSpec((_BLOCK, _N), lambda i: (i, 0))
    out_shape = jax.ShapeDtypeStruct((_DIM1, _N), jnp.float32)
    out1, out2 = pl.pallas_call(
        _spmm_body,
        grid=(_GRID,),
        in_specs=[pl.BlockSpec((1, _N), lambda i: (0, 0))],
        out_specs=(out_spec, out_spec),
        out_shape=(out_shape, out_shape),
    )(row0)
    return (out1, out2)
